# baseline (speedup 1.0000x reference)
import functools

import jax
import jax.numpy as jnp
from jax import lax
from jax.experimental import pallas as pl
from jax.experimental.pallas import tpu as pltpu
from jax.experimental.pallas import tpu_sc as plsc

_LANES = 16
_NBUF = 5
_BLOCK_ROWS = 128


def _build_gather(n_rows, n_pos, d, n_workers):
    assert n_rows % (n_workers * _BLOCK_ROWS) == 0
    rows_per_worker = n_rows // n_workers
    blocks_per_worker = rows_per_worker // _BLOCK_ROWS
    block_elems = _BLOCK_ROWS * d
    groups_per_block = _BLOCK_ROWS // _LANES

    mesh = plsc.VectorSubcoreMesh(core_axis_name="c", subcore_axis_name="s")

    @functools.partial(
        pl.kernel,
        mesh=mesh,
        compiler_params=pltpu.CompilerParams(needs_layout_passes=False),
        out_type=jax.ShapeDtypeStruct((n_rows * d,), jnp.float32),
        scratch_types=[
            pltpu.VMEM((rows_per_worker,), jnp.int32),
            pltpu.VMEM((n_pos * d,), jnp.float32),
        ]
        + [pltpu.VMEM((block_elems,), jnp.float32) for _ in range(_NBUF)]
        + [pltpu.SemaphoreType.DMA for _ in range(_NBUF)],
    )
    def gather_kernel(idx_hbm, pattern_hbm, out_hbm, idx_v, tbl_v, *bufs_and_sems):
        rows = bufs_and_sems[:_NBUF]
        ssem = bufs_and_sems[_NBUF : 2 * _NBUF]
        num_cores = lax.axis_size("c")
        wid = lax.axis_index("s") * num_cores + lax.axis_index("c")

        pltpu.sync_copy(pattern_hbm, tbl_v)
        pltpu.sync_copy(idx_hbm.at[pl.ds(wid * rows_per_worker, rows_per_worker)], idx_v)

        def mod_body(r, carry):
            sl = pl.ds(r * _LANES, _LANES)
            idx_v[sl] = lax.rem(idx_v[sl], n_pos)
            return carry

        lax.fori_loop(0, rows_per_worker // _LANES, mod_body, 0)

        out_base = wid * rows_per_worker
        iota = lax.iota(jnp.int32, _LANES)
        iota_d = iota * d

        def materialize(j, buf):
            def group_body(grp, carry):
                iv = idx_v[pl.ds(j * _BLOCK_ROWS + grp * _LANES, _LANES)]
                gidx = iv * d
                offs = iota_d + grp * (_LANES * d)
                one = jnp.ones((_LANES,), jnp.int32)
                for _c in range(d):
                    g = plsc.load_gather(tbl_v, [gidx])
                    plsc.store_scatter(buf, [offs], g)
                    gidx = gidx + one
                    offs = offs + one
                return carry

            lax.fori_loop(0, groups_per_block, group_body, 0)

        def scatter_copy(j, b):
            dst = out_hbm.at[pl.ds((out_base + j * _BLOCK_ROWS) * d, block_elems)]
            return pltpu.make_async_copy(rows[b], dst, ssem[b])

        def ring_body(g, carry):
            for b in range(_NBUF):
                j = g * _NBUF + b

                @pl.when(j >= _NBUF)
                def _():
                    scatter_copy(j - _NBUF, b).wait()

                materialize(j, rows[b])
                scatter_copy(j, b).start()
            return carry

        lax.fori_loop(0, blocks_per_worker // _NBUF, ring_body, 0)

        for t in range(blocks_per_worker - _NBUF, blocks_per_worker):
            scatter_copy(t, t % _NBUF).wait()

    return gather_kernel


def kernel(rec_current, visited_time, pattern):
    b, s = rec_current.shape
    n_pos, d = pattern.shape
    n_rows = b * s
    gather = _build_gather(n_rows, n_pos, d, n_workers=32)
    out = gather(visited_time.reshape(n_rows), pattern.reshape(n_pos * d))
    return out.reshape(b, s, d)


# speedup vs baseline: 2.3903x; 2.3903x over previous
import functools

import jax
import jax.numpy as jnp
from jax import lax
from jax.experimental import pallas as pl
from jax.experimental.pallas import tpu as pltpu
from jax.experimental.pallas import tpu_sc as plsc

_LANES = 16
_NBUF = 5
_BLOCK_ROWS = 128


def _build_gather(n_rows, n_pos, d, n_workers):
    assert n_rows % (n_workers * _BLOCK_ROWS) == 0
    rows_per_worker = n_rows // n_workers
    blocks_per_worker = rows_per_worker // _BLOCK_ROWS
    block_elems = _BLOCK_ROWS * d
    groups_per_block = _BLOCK_ROWS // _LANES

    mesh = plsc.VectorSubcoreMesh(core_axis_name="c", subcore_axis_name="s")

    @functools.partial(
        pl.kernel,
        mesh=mesh,
        compiler_params=pltpu.CompilerParams(needs_layout_passes=False),
        out_type=jax.ShapeDtypeStruct((n_rows * d,), jnp.float32),
        scratch_types=[
            pltpu.VMEM((rows_per_worker,), jnp.int32),
            pltpu.VMEM((n_pos * d,), jnp.float32),
        ]
        + [pltpu.VMEM((block_elems,), jnp.float32) for _ in range(_NBUF)]
        + [pltpu.SemaphoreType.DMA for _ in range(_NBUF)],
    )
    def gather_kernel(idx_hbm, pattern_hbm, out_hbm, idx_v, tbl_v, *bufs_and_sems):
        rows = bufs_and_sems[:_NBUF]
        ssem = bufs_and_sems[_NBUF : 2 * _NBUF]
        num_cores = lax.axis_size("c")
        wid = lax.axis_index("s") * num_cores + lax.axis_index("c")

        pltpu.sync_copy(pattern_hbm, tbl_v)
        pltpu.sync_copy(idx_hbm.at[pl.ds(wid * rows_per_worker, rows_per_worker)], idx_v)

        def mod_body(r, carry):
            sl = pl.ds(r * _LANES, _LANES)
            idx_v[sl] = lax.rem(idx_v[sl], n_pos)
            return carry

        lax.fori_loop(0, rows_per_worker // _LANES, mod_body, 0)

        out_base = wid * rows_per_worker
        iota = lax.iota(jnp.int32, _LANES)
        iota_d = iota * d

        def materialize(j, buf):
            def group_body(grp, carry):
                iv = idx_v[pl.ds(j * _BLOCK_ROWS + grp * _LANES, _LANES)]
                ivd = iv * d
                offs0 = iota_d + grp * (_LANES * d)

                @plsc.parallel_loop(0, d, unroll=8)
                def _cbody(c):
                    g = plsc.load_gather(tbl_v, [ivd + c])
                    plsc.store_scatter(buf, [offs0 + c], g)

                return carry

            lax.fori_loop(0, groups_per_block, group_body, 0)

        def scatter_copy(j, b):
            dst = out_hbm.at[pl.ds((out_base + j * _BLOCK_ROWS) * d, block_elems)]
            return pltpu.make_async_copy(rows[b], dst, ssem[b])

        def ring_body(g, carry):
            for b in range(_NBUF):
                j = g * _NBUF + b

                @pl.when(j >= _NBUF)
                def _():
                    scatter_copy(j - _NBUF, b).wait()

                materialize(j, rows[b])
                scatter_copy(j, b).start()
            return carry

        lax.fori_loop(0, blocks_per_worker // _NBUF, ring_body, 0)

        for t in range(blocks_per_worker - _NBUF, blocks_per_worker):
            scatter_copy(t, t % _NBUF).wait()

    return gather_kernel


def kernel(rec_current, visited_time, pattern):
    b, s = rec_current.shape
    n_pos, d = pattern.shape
    n_rows = b * s
    gather = _build_gather(n_rows, n_pos, d, n_workers=32)
    out = gather(visited_time.reshape(n_rows), pattern.reshape(n_pos * d))
    return out.reshape(b, s, d)


# speedup vs baseline: 5.3673x; 2.2454x over previous
import functools

import jax
import jax.numpy as jnp
from jax import lax
from jax.experimental import pallas as pl
from jax.experimental.pallas import tpu as pltpu
from jax.experimental.pallas import tpu_sc as plsc

_LANES = 16
_NBUF = 5
_BLOCK_ROWS = 128


def _build_gather(n_rows, n_pos, d, n_workers):
    assert n_rows % (n_workers * _BLOCK_ROWS) == 0
    rows_per_worker = n_rows // n_workers
    blocks_per_worker = rows_per_worker // _BLOCK_ROWS
    block_elems = _BLOCK_ROWS * d
    groups_per_block = _BLOCK_ROWS // _LANES

    mesh = plsc.VectorSubcoreMesh(core_axis_name="c", subcore_axis_name="s")

    @functools.partial(
        pl.kernel,
        mesh=mesh,
        compiler_params=pltpu.CompilerParams(needs_layout_passes=False),
        out_type=jax.ShapeDtypeStruct((n_rows * d,), jnp.float32),
        scratch_types=[
            pltpu.VMEM((rows_per_worker,), jnp.int32),
            pltpu.VMEM((n_pos * d,), jnp.float32),
        ]
        + [pltpu.VMEM((block_elems,), jnp.float32) for _ in range(_NBUF)]
        + [pltpu.SemaphoreType.DMA for _ in range(_NBUF)],
    )
    def gather_kernel(idx_hbm, pattern_hbm, out_hbm, idx_v, tbl_v, *bufs_and_sems):
        rows = bufs_and_sems[:_NBUF]
        ssem = bufs_and_sems[_NBUF : 2 * _NBUF]
        num_cores = lax.axis_size("c")
        wid = lax.axis_index("s") * num_cores + lax.axis_index("c")

        pltpu.sync_copy(pattern_hbm, tbl_v)
        pltpu.sync_copy(idx_hbm.at[pl.ds(wid * rows_per_worker, rows_per_worker)], idx_v)

        def mod_body(r, carry):
            sl = pl.ds(r * _LANES, _LANES)
            idx_v[sl] = lax.rem(idx_v[sl], n_pos)
            return carry

        lax.fori_loop(0, rows_per_worker // _LANES, mod_body, 0)

        out_base = wid * rows_per_worker
        iota = lax.iota(jnp.int32, _LANES)
        coffs = [iota + c8 * _LANES for c8 in range(d // _LANES)]
        lanes_of = [jnp.full((_LANES,), p, jnp.int32) for p in range(_LANES)]

        def materialize(j, buf):
            @plsc.parallel_loop(0, groups_per_block, unroll=1)
            def _group_body(grp):
                iv = idx_v[pl.ds(j * _BLOCK_ROWS + grp * _LANES, _LANES)]
                ivd = iv * d
                gbase = grp * (_LANES * d)
                for p in range(_LANES):
                    rb = jnp.take_along_axis(ivd, lanes_of[p], axis=0)
                    for c8 in range(d // _LANES):
                        g = plsc.load_gather(tbl_v, [rb + coffs[c8]])
                        buf[pl.ds(gbase + p * d + c8 * _LANES, _LANES)] = g

        def scatter_copy(j, b):
            dst = out_hbm.at[pl.ds((out_base + j * _BLOCK_ROWS) * d, block_elems)]
            return pltpu.make_async_copy(rows[b], dst, ssem[b])

        def ring_body(g, carry):
            for b in range(_NBUF):
                j = g * _NBUF + b

                @pl.when(j >= _NBUF)
                def _():
                    scatter_copy(j - _NBUF, b).wait()

                materialize(j, rows[b])
                scatter_copy(j, b).start()
            return carry

        lax.fori_loop(0, blocks_per_worker // _NBUF, ring_body, 0)

        for t in range(blocks_per_worker - _NBUF, blocks_per_worker):
            scatter_copy(t, t % _NBUF).wait()

    return gather_kernel


def kernel(rec_current, visited_time, pattern):
    b, s = rec_current.shape
    n_pos, d = pattern.shape
    n_rows = b * s
    gather = _build_gather(n_rows, n_pos, d, n_workers=32)
    out = gather(visited_time.reshape(n_rows), pattern.reshape(n_pos * d))
    return out.reshape(b, s, d)


# speedup vs baseline: 5.8881x; 1.0970x over previous
import functools

import jax
import jax.numpy as jnp
from jax import lax
from jax.experimental import pallas as pl
from jax.experimental.pallas import tpu as pltpu
from jax.experimental.pallas import tpu_sc as plsc

_LANES = 16
_NBUF = 2
_BLOCK_ROWS = 128


def _build_gather(n_rows, n_pos, d, n_workers):
    assert n_rows % (n_workers * _BLOCK_ROWS) == 0
    rows_per_worker = n_rows // n_workers
    blocks_per_worker = rows_per_worker // _BLOCK_ROWS
    block_elems = _BLOCK_ROWS * d
    groups_per_block = _BLOCK_ROWS // _LANES

    mesh = plsc.VectorSubcoreMesh(core_axis_name="c", subcore_axis_name="s")

    @functools.partial(
        pl.kernel,
        mesh=mesh,
        compiler_params=pltpu.CompilerParams(needs_layout_passes=False),
        out_type=jax.ShapeDtypeStruct((n_rows * d,), jnp.float32),
        scratch_types=[
            pltpu.VMEM((rows_per_worker,), jnp.int32),
            pltpu.VMEM((n_pos * d,), jnp.float32),
        ]
        + [pltpu.VMEM((block_elems,), jnp.float32) for _ in range(_NBUF)]
        + [pltpu.SemaphoreType.DMA for _ in range(_NBUF)],
    )
    def gather_kernel(idx_hbm, pattern_hbm, out_hbm, idx_v, tbl_v, *bufs_and_sems):
        rows = bufs_and_sems[:_NBUF]
        ssem = bufs_and_sems[_NBUF : 2 * _NBUF]
        num_cores = lax.axis_size("c")
        wid = lax.axis_index("s") * num_cores + lax.axis_index("c")

        pltpu.sync_copy(pattern_hbm, tbl_v)
        pltpu.sync_copy(idx_hbm.at[pl.ds(wid * rows_per_worker, rows_per_worker)], idx_v)

        @plsc.parallel_loop(0, rows_per_worker // _LANES, unroll=4)
        def _mod_body(r):
            sl = pl.ds(r * _LANES, _LANES)
            idx_v[sl] = lax.rem(idx_v[sl], n_pos) * d

        out_base = wid * rows_per_worker
        iota = lax.iota(jnp.int32, _LANES)
        coffs = [iota + c8 * _LANES for c8 in range(d // _LANES)]
        lanes_of = [jnp.full((_LANES,), p, jnp.int32) for p in range(_LANES)]

        def materialize(j, buf):
            @plsc.parallel_loop(0, groups_per_block, unroll=1)
            def _group_body(grp):
                ivd = idx_v[pl.ds(j * _BLOCK_ROWS + grp * _LANES, _LANES)]
                gbase = grp * (_LANES * d)
                for p in range(_LANES):
                    rb = jnp.take_along_axis(ivd, lanes_of[p], axis=0)
                    for c8 in range(d // _LANES):
                        g = plsc.load_gather(tbl_v, [rb + coffs[c8]])
                        buf[pl.ds(gbase + p * d + c8 * _LANES, _LANES)] = g

        def scatter_copy(j, b):
            dst = out_hbm.at[pl.ds((out_base + j * _BLOCK_ROWS) * d, block_elems)]
            return pltpu.make_async_copy(rows[b], dst, ssem[b])

        def ring_body(g, carry):
            for b in range(_NBUF):
                j = g * _NBUF + b

                @pl.when(j >= _NBUF)
                def _():
                    scatter_copy(j - _NBUF, b).wait()

                materialize(j, rows[b])
                scatter_copy(j, b).start()
            return carry

        lax.fori_loop(0, blocks_per_worker // _NBUF, ring_body, 0)

        for t in range(blocks_per_worker - _NBUF, blocks_per_worker):
            scatter_copy(t, t % _NBUF).wait()

    return gather_kernel


def kernel(rec_current, visited_time, pattern):
    b, s = rec_current.shape
    n_pos, d = pattern.shape
    n_rows = b * s
    gather = _build_gather(n_rows, n_pos, d, n_workers=32)
    out = gather(visited_time.reshape(n_rows), pattern.reshape(n_pos * d))
    return out.reshape(b, s, d)


# speedup vs baseline: 12.2301x; 2.0771x over previous
import functools

import jax
import jax.numpy as jnp
from jax import lax
from jax.experimental import pallas as pl
from jax.experimental.pallas import tpu as pltpu
from jax.experimental.pallas import tpu_sc as plsc

_LANES = 16
_NBUF = 2
_BLOCK_ROWS = 128


def _build_gather(n_rows, n_pos, d, n_workers):
    assert n_rows % (n_workers * _BLOCK_ROWS) == 0
    rows_per_worker = n_rows // n_workers
    blocks_per_worker = rows_per_worker // _BLOCK_ROWS
    block_elems = _BLOCK_ROWS * d
    groups_per_block = _BLOCK_ROWS // _LANES

    mesh = plsc.VectorSubcoreMesh(core_axis_name="c", subcore_axis_name="s")

    @functools.partial(
        pl.kernel,
        mesh=mesh,
        compiler_params=pltpu.CompilerParams(needs_layout_passes=False),
        out_type=jax.ShapeDtypeStruct((n_rows * d,), jnp.float32),
        scratch_types=[
            pltpu.VMEM((rows_per_worker + _LANES,), jnp.int32),
            pltpu.VMEM((n_pos * d,), jnp.float32),
        ]
        + [pltpu.VMEM((block_elems,), jnp.float32) for _ in range(_NBUF)]
        + [pltpu.SemaphoreType.DMA for _ in range(_NBUF)],
    )
    def gather_kernel(idx_hbm, pattern_hbm, out_hbm, idx_v, tbl_v, *bufs_and_sems):
        rows = bufs_and_sems[:_NBUF]
        ssem = bufs_and_sems[_NBUF : 2 * _NBUF]
        num_cores = lax.axis_size("c")
        wid = lax.axis_index("s") * num_cores + lax.axis_index("c")

        pltpu.sync_copy(pattern_hbm, tbl_v)
        pltpu.sync_copy(
            idx_hbm.at[pl.ds(wid * rows_per_worker, rows_per_worker)],
            idx_v.at[pl.ds(0, rows_per_worker)],
        )

        @plsc.parallel_loop(0, rows_per_worker // _LANES, unroll=4)
        def _mod_body(r):
            sl = pl.ds(r * _LANES, _LANES)
            idx_v[sl] = lax.rem(idx_v[sl], n_pos) * d

        out_base = wid * rows_per_worker
        iota = lax.iota(jnp.int32, _LANES)
        lane0 = jnp.zeros((_LANES,), jnp.int32)
        tbl_slices = [
            tbl_v.at[pl.ds(c8 * _LANES, n_pos * d - c8 * _LANES)]
            for c8 in range(d // _LANES)
        ]

        def materialize(j, buf):
            @plsc.parallel_loop(0, _BLOCK_ROWS, unroll=4)
            def _row_body(r):
                iv16 = idx_v[pl.ds(j * _BLOCK_ROWS + r, _LANES)]
                rb = jnp.take_along_axis(iv16, lane0, axis=0)
                rbio = rb + iota
                gs = [
                    plsc.load_gather(tbl_slices[c8], [rbio])
                    for c8 in range(d // _LANES)
                ]
                for c8 in range(d // _LANES):
                    buf[pl.ds(r * d + c8 * _LANES, _LANES)] = gs[c8]

        def scatter_copy(j, b):
            dst = out_hbm.at[pl.ds((out_base + j * _BLOCK_ROWS) * d, block_elems)]
            return pltpu.make_async_copy(rows[b], dst, ssem[b])

        def ring_body(g, carry):
            for b in range(_NBUF):
                j = g * _NBUF + b

                @pl.when(j >= _NBUF)
                def _():
                    scatter_copy(j - _NBUF, b).wait()

                materialize(j, rows[b])
                scatter_copy(j, b).start()
            return carry

        lax.fori_loop(0, blocks_per_worker // _NBUF, ring_body, 0)

        for t in range(blocks_per_worker - _NBUF, blocks_per_worker):
            scatter_copy(t, t % _NBUF).wait()

    return gather_kernel


def kernel(rec_current, visited_time, pattern):
    b, s = rec_current.shape
    n_pos, d = pattern.shape
    n_rows = b * s
    gather = _build_gather(n_rows, n_pos, d, n_workers=32)
    out = gather(visited_time.reshape(n_rows), pattern.reshape(n_pos * d))
    return out.reshape(b, s, d)
